# baseline (device time: 22359 ns/iter reference)
import jax
import jax.numpy as jnp
from jax import lax
from jax.experimental import pallas as pl
from jax.experimental.pallas import tpu as pltpu

N_DEV = 32
M_PER = 32
N_COLS = 1024


def kernel(x, w_mat):
    def body(x_ref, w_ref, out_ref, acc_ref, x3_ref,
             send1_ref, send1f_ref, recv1_ref, recv1f_ref,
             send2_ref, recv2_ref,
             send3_ref, recv3_ref,
             s1_sem, r1_sems, s2_sem, r2_sem, s3_sem, r3_sem):
        p = lax.axis_index("i")
        zi = p >> 3
        q = p & 7
        q_hi = (p >> 2) & 1
        q_lo = p & 3
        base_sq = (p >> 2) << 2

        barrier = pltpu.get_barrier_semaphore()
        partners = (
            [base_sq + ((q_lo + d) & 3) for d in (1, 2, 3)]
            + [(((zi + d) & 3) << 3) + q for d in (1, 2, 3)]
            + [p ^ 4]
        )
        for t in partners:
            pl.semaphore_signal(
                barrier, inc=1, device_id=(t,),
                device_id_type=pl.DeviceIdType.MESH,
            )

        x3_ref[:] = x_ref[:].astype(jnp.bfloat16).reshape(N_DEV, M_PER, 32)
        wb = w_ref[:].astype(jnp.bfloat16)

        rdmas1 = {}
        for k in (1, 2, 3, 4):
            zt = (zi + k) & 3
            pg = jnp.dot(
                x3_ref[pl.ds(8 * zt, 8)].reshape(8 * M_PER, 32),
                wb,
                preferred_element_type=jnp.float32,
            ).reshape(2, 4, M_PER, N_COLS)
            for lc in range(4):
                for b in range(2):
                    send1_ref[lc * 8 + 2 * zt + b] = pg[b, lc].astype(
                        jnp.bfloat16
                    )
                    send1f_ref[lc * 8 + 2 * zt + b] = pg[b, lc].astype(
                        jnp.float8_e4m3fn
                    )
            if k == 1:
                pl.semaphore_wait(barrier, 7)
            flows = [
                ('x', q_lo ^ 1,
                 send1_ref.at[pl.ds((q_lo ^ 1) * 8 + 2 * zt, 2)],
                 recv1_ref.at[pl.ds((k - 1) * 2, 2)]),
                ('d', q_lo ^ 2,
                 send1f_ref.at[pl.ds((q_lo ^ 2) * 8 + 2 * zt, 2)],
                 recv1f_ref.at[pl.ds((k - 1) * 2, 2)]),
                ('y', 3 - q_lo,
                 send1f_ref.at[pl.ds((3 - q_lo) * 8 + 2 * zt, 2)],
                 recv1f_ref.at[pl.ds(8 + (k - 1) * 2, 2)]),
            ]
            for fl, t, src_r, dst_r in flows:
                rdma = pltpu.make_async_remote_copy(
                    src_ref=src_r,
                    dst_ref=dst_r,
                    send_sem=s1_sem,
                    recv_sem=r1_sems.at[k - 1],
                    device_id=(base_sq + t,),
                    device_id_type=pl.DeviceIdType.MESH,
                )
                rdma.start()
                rdmas1[(fl, k)] = rdma
            k01 = jnp.where(q_lo == 0, pg[:, 0], pg[:, 1])
            k23 = jnp.where(q_lo == 2, pg[:, 2], pg[:, 3])
            acc_ref[pl.ds(2 * zt, 2)] = jnp.where(q_lo < 2, k01, k23)

        rdmas2 = {}
        for k in (1, 2, 3):
            zt = (zi + k) & 3
            for fl in ('x', 'd', 'y'):
                rdmas1[(fl, k)].wait_recv()
            for b in range(2):
                g = (k - 1) * 2 + b
                send2_ref[g] = (
                    acc_ref[2 * zt + b]
                    + recv1_ref[g].astype(jnp.float32)
                    + recv1f_ref[g].astype(jnp.float32)
                    + recv1f_ref[8 + g].astype(jnp.float32)
                ).astype(jnp.bfloat16)
            rdma = pltpu.make_async_remote_copy(
                src_ref=send2_ref.at[pl.ds((k - 1) * 2, 2)],
                dst_ref=recv2_ref.at[pl.ds((3 - k) * 2, 2)],
                send_sem=s2_sem,
                recv_sem=r2_sem,
                device_id=((zt << 3) + q,),
                device_id_type=pl.DeviceIdType.MESH,
            )
            rdma.start()
            rdmas2[k] = rdma

        for fl in ('x', 'd', 'y'):
            rdmas1[(fl, 4)].wait_recv()
        for b in range(2):
            j = 2 * zi + b
            g = 6 + b
            acc_ref[j] = acc_ref[j] + (
                recv1_ref[g].astype(jnp.float32)
                + recv1f_ref[g].astype(jnp.float32)
                + recv1f_ref[8 + g].astype(jnp.float32)
            )

        for k in (1, 2, 3):
            rdmas2[k].wait_recv()
        b_send = 1 - q_hi
        send3_ref[:] = (
            acc_ref[2 * zi + b_send]
            + recv2_ref[b_send].astype(jnp.float32)
            + recv2_ref[2 + b_send].astype(jnp.float32)
            + recv2_ref[4 + b_send].astype(jnp.float32)
        ).astype(jnp.bfloat16)
        rdma3 = pltpu.make_async_remote_copy(
            src_ref=send3_ref,
            dst_ref=recv3_ref,
            send_sem=s3_sem,
            recv_sem=r3_sem,
            device_id=(p ^ 4,),
            device_id_type=pl.DeviceIdType.MESH,
        )
        rdma3.start()
        t_keep = (
            acc_ref[2 * zi + q_hi]
            + recv2_ref[q_hi].astype(jnp.float32)
            + recv2_ref[2 + q_hi].astype(jnp.float32)
            + recv2_ref[4 + q_hi].astype(jnp.float32)
        )
        rdma3.wait_recv()
        out_ref[:] = jnp.maximum(t_keep + recv3_ref[:].astype(jnp.float32), 0.0)

        for r in rdmas1.values():
            r.wait_send()
        for r in rdmas2.values():
            r.wait_send()
        rdma3.wait_send()

    return pl.pallas_call(
        body,
        out_shape=jax.ShapeDtypeStruct((M_PER, N_COLS), jnp.float32),
        in_specs=[
            pl.BlockSpec(memory_space=pltpu.VMEM),
            pl.BlockSpec(memory_space=pltpu.VMEM),
        ],
        out_specs=pl.BlockSpec(memory_space=pltpu.VMEM),
        scratch_shapes=[
            pltpu.VMEM((8, M_PER, N_COLS), jnp.float32),
            pltpu.VMEM((N_DEV, M_PER, 32), jnp.bfloat16),
            pltpu.VMEM((N_DEV, M_PER, N_COLS), jnp.bfloat16),
            pltpu.VMEM((N_DEV, M_PER, N_COLS), jnp.float8_e4m3fn),
            pltpu.VMEM((8, M_PER, N_COLS), jnp.bfloat16),
            pltpu.VMEM((16, M_PER, N_COLS), jnp.float8_e4m3fn),
            pltpu.VMEM((6, M_PER, N_COLS), jnp.bfloat16),
            pltpu.VMEM((6, M_PER, N_COLS), jnp.bfloat16),
            pltpu.VMEM((M_PER, N_COLS), jnp.bfloat16),
            pltpu.VMEM((M_PER, N_COLS), jnp.bfloat16),
            pltpu.SemaphoreType.DMA,
            pltpu.SemaphoreType.DMA((4,)),
            pltpu.SemaphoreType.DMA,
            pltpu.SemaphoreType.DMA,
            pltpu.SemaphoreType.DMA,
            pltpu.SemaphoreType.DMA,
        ],
        compiler_params=pltpu.CompilerParams(collective_id=0),
    )(x, w_mat)


# device time: 22321 ns/iter; 1.0017x vs baseline; 1.0017x over previous
import jax
import jax.numpy as jnp
from jax import lax
from jax.experimental import pallas as pl
from jax.experimental.pallas import tpu as pltpu

N_DEV = 32
M_PER = 32
N_COLS = 1024


def kernel(x, w_mat):
    def body(x_ref, w_ref, out_ref, acc_ref, x3_ref,
             send1_ref, send1f_ref, recv1_ref, recv1f_ref,
             send2_ref, recv2_ref,
             send3_ref, recv3_ref,
             s1_sem, r1_sems, s2_sem, r2_sem, s3_sem, r3_sem):
        p = lax.axis_index("i")
        zi = p >> 3
        q = p & 7
        q_hi = (p >> 2) & 1
        q_lo = p & 3
        base_sq = (p >> 2) << 2

        barrier = pltpu.get_barrier_semaphore()
        partners = (
            [base_sq + ((q_lo + d) & 3) for d in (1, 2, 3)]
            + [(((zi + d) & 3) << 3) + q for d in (1, 2, 3)]
            + [p ^ 4]
        )
        for t in partners:
            pl.semaphore_signal(
                barrier, inc=1, device_id=(t,),
                device_id_type=pl.DeviceIdType.MESH,
            )

        x3_ref[:] = x_ref[:].astype(jnp.bfloat16).reshape(N_DEV, M_PER, 32)
        wb = w_ref[:].astype(jnp.bfloat16)

        rdmas1 = {}
        for k in (1, 2, 3, 4):
            zt = (zi + k) & 3
            pg = jnp.dot(
                x3_ref[pl.ds(8 * zt, 8)].reshape(8 * M_PER, 32),
                wb,
                preferred_element_type=jnp.float32,
            ).reshape(2, 4, M_PER, N_COLS)
            for lc in range(4):
                for b in range(2):
                    send1_ref[lc * 8 + 2 * zt + b] = pg[b, lc].astype(
                        jnp.bfloat16
                    )
                    send1f_ref[lc * 8 + 2 * zt + b] = pg[b, lc].astype(
                        jnp.float8_e4m3fn
                    )
            if k == 1:
                pl.semaphore_wait(barrier, 7)
            for d in (1, 2, 3):
                t = (q_lo + d) & 3
                if d == 2:
                    src_r = send1f_ref.at[pl.ds(t * 8 + 2 * zt, 2)]
                    dst_r = recv1f_ref.at[pl.ds((k - 1) * 2, 2)]
                else:
                    src_r = send1_ref.at[pl.ds(t * 8 + 2 * zt, 2)]
                    dst_r = recv1_ref.at[pl.ds((3 - d) * 8 + (k - 1) * 2, 2)]
                rdma = pltpu.make_async_remote_copy(
                    src_ref=src_r,
                    dst_ref=dst_r,
                    send_sem=s1_sem,
                    recv_sem=r1_sems.at[k - 1],
                    device_id=(base_sq + t,),
                    device_id_type=pl.DeviceIdType.MESH,
                )
                rdma.start()
                rdmas1[(d, k)] = rdma
            k01 = jnp.where(q_lo == 0, pg[:, 0], pg[:, 1])
            k23 = jnp.where(q_lo == 2, pg[:, 2], pg[:, 3])
            acc_ref[pl.ds(2 * zt, 2)] = jnp.where(q_lo < 2, k01, k23)

        rdmas2 = {}
        for k in (1, 2, 3):
            zt = (zi + k) & 3
            for d in (1, 2, 3):
                rdmas1[(d, k)].wait_recv()
            for b in range(2):
                g = (k - 1) * 2 + b
                send2_ref[g] = (
                    acc_ref[2 * zt + b]
                    + recv1_ref[g].astype(jnp.float32)
                    + recv1f_ref[g].astype(jnp.float32)
                    + recv1_ref[16 + g].astype(jnp.float32)
                ).astype(jnp.bfloat16)
            rdma = pltpu.make_async_remote_copy(
                src_ref=send2_ref.at[pl.ds((k - 1) * 2, 2)],
                dst_ref=recv2_ref.at[pl.ds((3 - k) * 2, 2)],
                send_sem=s2_sem,
                recv_sem=r2_sem,
                device_id=((zt << 3) + q,),
                device_id_type=pl.DeviceIdType.MESH,
            )
            rdma.start()
            rdmas2[k] = rdma

        for d in (1, 2, 3):
            rdmas1[(d, 4)].wait_recv()
        for b in range(2):
            j = 2 * zi + b
            g = 6 + b
            acc_ref[j] = acc_ref[j] + (
                recv1_ref[g].astype(jnp.float32)
                + recv1f_ref[g].astype(jnp.float32)
                + recv1_ref[16 + g].astype(jnp.float32)
            )

        for k in (1, 2, 3):
            rdmas2[k].wait_recv()
        b_send = 1 - q_hi
        send3_ref[:] = (
            acc_ref[2 * zi + b_send]
            + recv2_ref[b_send].astype(jnp.float32)
            + recv2_ref[2 + b_send].astype(jnp.float32)
            + recv2_ref[4 + b_send].astype(jnp.float32)
        ).astype(jnp.bfloat16)
        rdma3 = pltpu.make_async_remote_copy(
            src_ref=send3_ref,
            dst_ref=recv3_ref,
            send_sem=s3_sem,
            recv_sem=r3_sem,
            device_id=(p ^ 4,),
            device_id_type=pl.DeviceIdType.MESH,
        )
        rdma3.start()
        t_keep = (
            acc_ref[2 * zi + q_hi]
            + recv2_ref[q_hi].astype(jnp.float32)
            + recv2_ref[2 + q_hi].astype(jnp.float32)
            + recv2_ref[4 + q_hi].astype(jnp.float32)
        )
        rdma3.wait_recv()
        out_ref[:] = jnp.maximum(t_keep + recv3_ref[:].astype(jnp.float32), 0.0)

        for r in rdmas1.values():
            r.wait_send()
        for r in rdmas2.values():
            r.wait_send()
        rdma3.wait_send()

    return pl.pallas_call(
        body,
        out_shape=jax.ShapeDtypeStruct((M_PER, N_COLS), jnp.float32),
        in_specs=[
            pl.BlockSpec(memory_space=pltpu.VMEM),
            pl.BlockSpec(memory_space=pltpu.VMEM),
        ],
        out_specs=pl.BlockSpec(memory_space=pltpu.VMEM),
        scratch_shapes=[
            pltpu.VMEM((8, M_PER, N_COLS), jnp.float32),
            pltpu.VMEM((N_DEV, M_PER, 32), jnp.bfloat16),
            pltpu.VMEM((N_DEV, M_PER, N_COLS), jnp.bfloat16),
            pltpu.VMEM((N_DEV, M_PER, N_COLS), jnp.float8_e4m3fn),
            pltpu.VMEM((24, M_PER, N_COLS), jnp.bfloat16),
            pltpu.VMEM((8, M_PER, N_COLS), jnp.float8_e4m3fn),
            pltpu.VMEM((6, M_PER, N_COLS), jnp.bfloat16),
            pltpu.VMEM((6, M_PER, N_COLS), jnp.bfloat16),
            pltpu.VMEM((M_PER, N_COLS), jnp.bfloat16),
            pltpu.VMEM((M_PER, N_COLS), jnp.bfloat16),
            pltpu.SemaphoreType.DMA,
            pltpu.SemaphoreType.DMA((4,)),
            pltpu.SemaphoreType.DMA,
            pltpu.SemaphoreType.DMA,
            pltpu.SemaphoreType.DMA,
            pltpu.SemaphoreType.DMA,
        ],
        compiler_params=pltpu.CompilerParams(collective_id=0),
    )(x, w_mat)
